# Initial kernel scaffold; baseline (speedup 1.0000x reference)
import jax
import jax.numpy as jnp
from jax.experimental import pallas as pl


def kernel(class_out, box_out, anchors):
    raise NotImplementedError("write your pallas kernel here")


# speedup vs baseline: 1.0905x; 1.0905x over previous
import functools

import jax
import jax.numpy as jnp
import numpy as np
from jax.experimental import pallas as pl
from jax.experimental.pallas import tpu as pltpu

_NUM_CLASSES = 90
_MAX_DET_POINTS = 5000
_MAX_DET = 100
_IOU_THRESH = 0.5
_CONF_THRESH = 0.3
_BBOX_XFORM_CLIP = float(np.log(1000.0 / 16.0))
_NEG = -1e30


def _nms_body(f_ref, o_ref):
  ty = f_ref[0]
  tx = f_ref[1]
  th = jnp.minimum(f_ref[2], _BBOX_XFORM_CLIP)
  tw = jnp.minimum(f_ref[3], _BBOX_XFORM_CLIP)
  ay1 = f_ref[4]
  ax1 = f_ref[5]
  ay2 = f_ref[6]
  ax2 = f_ref[7]
  logit = f_ref[8]
  clsf = f_ref[9]

  yca = (ay1 + ay2) / 2.0
  xca = (ax1 + ax2) / 2.0
  ha = ay2 - ay1
  wa = ax2 - ax1
  yc = ty * ha + yca
  xc = tx * wa + xca
  h = jnp.exp(th) * ha
  w = jnp.exp(tw) * wa
  off = clsf * 4096.0
  nb0 = (yc - h / 2.0) + off
  nb1 = (xc - w / 2.0) + off
  nb2 = (yc + h / 2.0) + off
  nb3 = (xc + w / 2.0) + off
  areas = (nb2 - nb0) * (nb3 - nb1)
  sc0 = jnp.where(logit > 0.5 * _NEG, jax.nn.sigmoid(logit), -1.0)

  bcap = sc0.shape
  iota = jax.lax.broadcasted_iota(jnp.int32, bcap, 1)
  lane6 = jax.lax.broadcasted_iota(jnp.int32, (bcap[0], 128), 1)

  def step(t, carry):
    sc, first = carry
    m = jnp.max(sc, axis=1, keepdims=True)
    hit = sc == m
    lidx = jnp.where(hit, iota, bcap[1])
    idxm = jnp.min(lidx, axis=1, keepdims=True)
    onehot = jnp.where(iota == idxm, 1.0, 0.0)

    def ext(a):
      return jnp.sum(a * onehot, axis=1, keepdims=True)

    e0 = ext(nb0)
    e1 = ext(nb1)
    e2 = ext(nb2)
    e3 = ext(nb3)
    ec = ext(clsf)

    yy1 = jnp.maximum(e0, nb0)
    xx1 = jnp.maximum(e1, nb1)
    yy2 = jnp.minimum(e2, nb2)
    xx2 = jnp.minimum(e3, nb3)
    inter = jnp.maximum(yy2 - yy1, 0.0) * jnp.maximum(xx2 - xx1, 0.0)
    area_b = (e2 - e0) * (e3 - e1)
    iou = inter / (area_b + areas - inter + 1e-8)
    sc = jnp.where(iou > _IOU_THRESH, -1.0, sc)

    eoff = ec * 4096.0
    cur = jnp.concatenate(
        [e0 - eoff, e1 - eoff, e2 - eoff, e3 - eoff, ec], axis=1)
    first = jnp.where(t == 0, cur, first)
    deg = m < 0.0
    row5 = jnp.where(deg, first, cur)
    score = jnp.where(m > _CONF_THRESH, m, 0.0)
    row = (
        jnp.where(lane6 == 0, row5[:, 0:1], 0.0)
        + jnp.where(lane6 == 1, row5[:, 1:2], 0.0)
        + jnp.where(lane6 == 2, row5[:, 2:3], 0.0)
        + jnp.where(lane6 == 3, row5[:, 3:4], 0.0)
        + jnp.where(lane6 == 4, score, 0.0)
        + jnp.where(lane6 == 5, row5[:, 4:5], 0.0)
    )
    o_ref[pl.ds(t, 1)] = row[None]
    return sc, first

  jax.lax.fori_loop(0, _MAX_DET, step, (sc0, jnp.zeros((bcap[0], 5))),
                    unroll=False)


def _run_nms(fields):
  b = fields.shape[1]
  out = pl.pallas_call(
      _nms_body,
      out_shape=jax.ShapeDtypeStruct((_MAX_DET, b, 128), jnp.float32),
      in_specs=[pl.BlockSpec(memory_space=pltpu.MemorySpace.VMEM)],
      out_specs=pl.BlockSpec(memory_space=pltpu.MemorySpace.VMEM),
  )(fields)
  return out[:, :, :6].transpose(1, 0, 2)


def kernel(class_out, box_out, anchors):
  b, n, c = class_out.shape
  flat = class_out.reshape(b, n * c)
  top_vals, top_idx = jax.lax.top_k(flat, _MAX_DET_POINTS)
  anchor_idx = top_idx // c
  cls_idx = top_idx % c
  box_sel = jnp.take_along_axis(box_out, anchor_idx[..., None], axis=1)
  anchors_sel = jnp.take(anchors, anchor_idx, axis=0)

  cap = 5120
  pad = cap - _MAX_DET_POINTS
  fields = jnp.stack([
      box_sel[..., 0], box_sel[..., 1], box_sel[..., 2], box_sel[..., 3],
      anchors_sel[..., 0], anchors_sel[..., 1], anchors_sel[..., 2],
      anchors_sel[..., 3], top_vals, cls_idx.astype(jnp.float32)
  ])
  padv = jnp.zeros((10, b, pad), jnp.float32).at[8].set(_NEG)
  fields = jnp.concatenate([fields, padv], axis=2)
  return _run_nms(fields)


# speedup vs baseline: 12.2445x; 11.2285x over previous
import functools

import jax
import jax.numpy as jnp
import numpy as np
from jax import lax
from jax.experimental import pallas as pl
from jax.experimental.pallas import tpu as pltpu
from jax.experimental.pallas import tpu_sc as plsc

_NUM_CLASSES = 90
_MAX_DET_POINTS = 5000
_MAX_DET = 100
_IOU_THRESH = 0.5
_CONF_THRESH = 0.3
_BBOX_XFORM_CLIP = float(np.log(1000.0 / 16.0))
_NEG = -1e30

_B = 8
_NPB = 1_800_000
_QN = _NPB // 4
_CHUNK = 18_000
_NCHUNK = _QN // _CHUNK
_VPC = _CHUNK // 16
_HBINS = 8192
_QBINS = _HBINS // 4
_WCAP = 3072
_CAP = 8192
_PCAP = _CAP // 4
_STAGE = 4 * _WCAP
_K = _MAX_DET_POINTS
_MINI32 = np.int32(-2147483648)


def _sc_body(flat, boxflat, anchors, out, inbuf, hist, vals, idxs, smallv,
             smalli, cval, cidx, aidxb, aidxa, fbufs,
             hist_sh, red_sh, qtot_sh, tf_sh, cnt_sh, vstage_sh, istage_sh,
             gsem):
  c = lax.axis_index("c")
  s = lax.axis_index("s")
  g = s // 4
  p = s % 4
  batch = c * 4 + g
  wtile = s
  base = batch * _NPB + p * _QN
  iota16 = lax.broadcasted_iota(jnp.int32, (16,), 0)
  ones16 = jnp.ones((16,), jnp.int32)

  def zero_hist(i, _):
    hist[pl.ds(i * 16, 16)] = jnp.zeros((16,), jnp.int32)
    return 0

  lax.fori_loop(0, _HBINS // 16, zero_hist, 0)

  def chunk_a(ci, _):
    pltpu.sync_copy(flat.at[pl.ds(base + ci * _CHUNK, _CHUNK)], inbuf)

    def vloop(i, _):
      v = inbuf[pl.ds(i * 16, 16)]
      bts = plsc.bitcast(v, jnp.int32)
      key = bts ^ (lax.shift_right_arithmetic(bts, 31) | _MINI32)
      bucket = lax.shift_right_logical(key, 19)
      plsc.addupdate_scatter(hist, [bucket], ones16)
      return 0

    lax.fori_loop(0, _VPC, vloop, 0)
    return 0

  lax.fori_loop(0, _NCHUNK, chunk_a, 0)

  pltpu.sync_copy(hist, hist_sh.at[pl.ds(wtile * _HBINS, _HBINS)])
  plsc.subcore_barrier()

  qbase = p * _QBINS
  pltpu.sync_copy(hist_sh.at[pl.ds(4 * g * _HBINS + qbase, _QBINS)],
                  hist.at[pl.ds(0, _QBINS)])
  for q in range(1, 4):
    pltpu.sync_copy(hist_sh.at[pl.ds((4 * g + q) * _HBINS + qbase, _QBINS)],
                    hist.at[pl.ds(_QBINS, _QBINS)])

    def addq(i, _):
      hist[pl.ds(i * 16, 16)] = (
          hist[pl.ds(i * 16, 16)] + hist[pl.ds(_QBINS + i * 16, 16)])
      return 0

    lax.fori_loop(0, _QBINS // 16, addq, 0)
  pltpu.sync_copy(hist.at[pl.ds(0, _QBINS)],
                  red_sh.at[pl.ds(g * _HBINS + qbase, _QBINS)])

  def qsum(i, a):
    return a + hist[pl.ds(i * 16, 16)]

  part16 = lax.fori_loop(0, _QBINS // 16, qsum, jnp.zeros((16,), jnp.int32))
  qtot = jnp.sum(part16)
  smalli[pl.ds(0, 16)] = jnp.full((16,), qtot, jnp.int32)
  pltpu.sync_copy(smalli.at[pl.ds(0, 128)],
                  qtot_sh.at[pl.ds((g * 4 + p) * 128, 128)])
  plsc.subcore_barrier()

  @pl.when(p == 0)
  def _threshold():
    pltpu.sync_copy(qtot_sh.at[pl.ds(g * 512, 512)], smalli)
    q0 = jnp.max(smalli[pl.ds(0 * 128, 16)])
    q1 = jnp.max(smalli[pl.ds(1 * 128, 16)])
    q2 = jnp.max(smalli[pl.ds(2 * 128, 16)])
    q3 = jnp.max(smalli[pl.ds(3 * 128, 16)])
    c3, c2, c1 = q3, q3 + q2, q3 + q2 + q1
    qq = jnp.where(c3 >= _K, 3, jnp.where(c2 >= _K, 2,
                                          jnp.where(c1 >= _K, 1, 0)))
    above = jnp.where(c3 >= _K, 0, jnp.where(c2 >= _K, q3,
                                             jnp.where(c1 >= _K, c2, c1)))
    pltpu.sync_copy(red_sh.at[pl.ds(g * _HBINS + qq * _QBINS, _QBINS)],
                    hist.at[pl.ds(0, _QBINS)])

    def scan(j, carry):
      accum, bsel, done = carry
      jj = _QBINS // 16 - 1 - j
      v = hist[pl.ds(jj * 16, 16)]
      rv = lax.rev(v, (0,))
      cs = plsc.cumsum(rv)
      tot = jnp.max(cs)
      mask = (accum + cs) >= _K
      found = jnp.logical_and(jnp.logical_not(done), (accum + tot) >= _K)
      lane = jnp.max(plsc.all_reduce_ffs(mask))
      bcand = qq * _QBINS + jj * 16 + 15 - lane
      bsel = jnp.where(found, bcand, bsel)
      done = jnp.logical_or(done, found)
      return accum + tot, bsel, done

    _, bsel, _ = lax.fori_loop(0, _QBINS // 16, scan,
                               (above, jnp.int32(0), False))
    bv = jnp.full((16,), bsel, jnp.int32)
    u = lax.shift_left(bv, 19)
    fb = jnp.where(u < 0, u ^ _MINI32, ~u)
    smallv[pl.ds(0, 16)] = plsc.bitcast(fb, jnp.float32)
    pltpu.sync_copy(smallv, tf_sh.at[pl.ds(g * 128, 128)])

  plsc.subcore_barrier()
  pltpu.sync_copy(tf_sh.at[pl.ds(g * 128, 128)], smallv)
  tv = smallv[pl.ds(0, 16)]

  def chunk_b(ci, off):
    pltpu.sync_copy(flat.at[pl.ds(base + ci * _CHUNK, _CHUNK)], inbuf)
    cbase = p * _QN + ci * _CHUNK

    def vloop(i, off):
      v = inbuf[pl.ds(i * 16, 16)]
      m = v >= tv
      iv = cbase + i * 16 + iota16
      plsc.store_compressed(vals.at[pl.ds(off, 16)], v, mask=m)
      plsc.store_compressed(idxs.at[pl.ds(off, 16)], iv, mask=m)
      return jnp.minimum(off + jnp.sum(m.astype(jnp.int32)), _WCAP - 160)

    return lax.fori_loop(0, _VPC, vloop, off)

  off = lax.fori_loop(0, _NCHUNK, chunk_b, jnp.int32(0))
  for k in range(8):
    vals[pl.ds(off + k * 16, 16)] = jnp.full((16,), _NEG, jnp.float32)
    idxs[pl.ds(off + k * 16, 16)] = jnp.zeros((16,), jnp.int32)
  offr = (off + 127) // 128 * 128
  smalli[pl.ds(0, 16)] = jnp.full((16,), offr, jnp.int32)
  pltpu.sync_copy(smalli.at[pl.ds(0, 128)],
                  cnt_sh.at[pl.ds((g * 4 + p) * 128, 128)])
  plsc.subcore_barrier()

  pltpu.sync_copy(cnt_sh.at[pl.ds(g * 512, 512)], smalli)
  n0 = jnp.max(smalli[pl.ds(0 * 128, 16)])
  n1 = jnp.max(smalli[pl.ds(1 * 128, 16)])
  n2 = jnp.max(smalli[pl.ds(2 * 128, 16)])
  n3 = jnp.max(smalli[pl.ds(3 * 128, 16)])
  myoff = jnp.where(p == 0, 0, jnp.where(p == 1, n0,
                                         jnp.where(p == 2, n0 + n1,
                                                   n0 + n1 + n2)))
  mr = n0 + n1 + n2 + n3

  for ph in range(4):
    @pl.when(p == ph)
    def _stage():
      moff = pl.multiple_of(g * _STAGE + myoff, 128)
      pltpu.sync_copy(vals, vstage_sh.at[pl.ds(moff, _WCAP)])
      pltpu.sync_copy(idxs, istage_sh.at[pl.ds(moff, _WCAP)])

    plsc.subcore_barrier()

  cb = p * _PCAP
  pltpu.sync_copy(vstage_sh.at[pl.ds(g * _STAGE + cb, _PCAP)], cval)
  pltpu.sync_copy(istage_sh.at[pl.ds(g * _STAGE + cb, _PCAP)], cidx)
  rowoff = batch * 20000

  def prep(k, _):
    pos = cb + k * 16 + iota16
    valid = pos < mr
    v = jnp.where(valid, cval[pl.ds(k * 16, 16)], _NEG)
    fi = jnp.where(valid, cidx[pl.ds(k * 16, 16)], 0)
    a = fi // _NUM_CLASSES
    cls = fi - a * _NUM_CLASSES
    for cdim in range(4):
      aidxb[pl.ds(cdim * _PCAP + k * 16, 16)] = (a + rowoff) * 4 + cdim
      aidxa[pl.ds(cdim * _PCAP + k * 16, 16)] = a * 4 + cdim
    fbufs[pl.ds(8 * _PCAP + k * 16, 16)] = v
    fbufs[pl.ds(9 * _PCAP + k * 16, 16)] = cls.astype(jnp.float32)
    return 0

  lax.fori_loop(0, _PCAP // 16, prep, 0)

  cops = []
  for cdim in range(4):
    for j in range(_PCAP // 128):
      o = cdim * _PCAP + j * 128
      cops.append(pltpu.async_copy(
          boxflat.at[aidxb.at[pl.ds(o, 128)]],
          fbufs.at[pl.ds(cdim * _PCAP + j * 128, 128)], gsem))
      cops.append(pltpu.async_copy(
          anchors.at[aidxa.at[pl.ds(o, 128)]],
          fbufs.at[pl.ds((4 + cdim) * _PCAP + j * 128, 128)], gsem))
  for cp in cops:
    cp.wait()

  for f in range(10):
    pltpu.sync_copy(
        fbufs.at[pl.ds(f * _PCAP, _PCAP)],
        out.at[pl.ds(f * (_B * _CAP) + batch * _CAP + cb, _PCAP)])


def _sc_select(flat, boxflat, anchors):
  mesh = plsc.VectorSubcoreMesh(core_axis_name="c", subcore_axis_name="s",
                                num_cores=2, num_subcores=16)
  return pl.kernel(
      _sc_body,
      out_type=jax.ShapeDtypeStruct((10 * _B * _CAP,), jnp.float32),
      mesh=mesh,
      compiler_params=pltpu.CompilerParams(needs_layout_passes=False),
      scratch_types=[
          pltpu.VMEM((_CHUNK,), jnp.float32),
          pltpu.VMEM((_HBINS,), jnp.int32),
          pltpu.VMEM((_WCAP,), jnp.float32),
          pltpu.VMEM((_WCAP,), jnp.int32),
          pltpu.VMEM((128,), jnp.float32),
          pltpu.VMEM((512,), jnp.int32),
          pltpu.VMEM((_PCAP,), jnp.float32),
          pltpu.VMEM((_PCAP,), jnp.int32),
          pltpu.VMEM((4 * _PCAP,), jnp.int32),
          pltpu.VMEM((4 * _PCAP,), jnp.int32),
          pltpu.VMEM((10 * _PCAP,), jnp.float32),
          pltpu.VMEM_SHARED((16 * _HBINS,), jnp.int32),
          pltpu.VMEM_SHARED((4 * _HBINS,), jnp.int32),
          pltpu.VMEM_SHARED((2048,), jnp.int32),
          pltpu.VMEM_SHARED((512,), jnp.float32),
          pltpu.VMEM_SHARED((2048,), jnp.int32),
          pltpu.VMEM_SHARED((4 * _STAGE,), jnp.float32),
          pltpu.VMEM_SHARED((4 * _STAGE,), jnp.int32),
          pltpu.SemaphoreType.DMA,
      ],
  )(flat, boxflat, anchors)


def _nms_body(f_ref, o_ref):
  ty = f_ref[0]
  tx = f_ref[1]
  th = jnp.minimum(f_ref[2], _BBOX_XFORM_CLIP)
  tw = jnp.minimum(f_ref[3], _BBOX_XFORM_CLIP)
  ay1 = f_ref[4]
  ax1 = f_ref[5]
  ay2 = f_ref[6]
  ax2 = f_ref[7]
  logit = f_ref[8]
  clsf = f_ref[9]

  yca = (ay1 + ay2) / 2.0
  xca = (ax1 + ax2) / 2.0
  ha = ay2 - ay1
  wa = ax2 - ax1
  yc = ty * ha + yca
  xc = tx * wa + xca
  h = jnp.exp(th) * ha
  w = jnp.exp(tw) * wa
  off = clsf * 4096.0
  nb0 = (yc - h / 2.0) + off
  nb1 = (xc - w / 2.0) + off
  nb2 = (yc + h / 2.0) + off
  nb3 = (xc + w / 2.0) + off
  areas = (nb2 - nb0) * (nb3 - nb1)
  sc0 = jnp.where(logit > 0.5 * _NEG, jax.nn.sigmoid(logit), -1.0)

  bcap = sc0.shape
  iota = jax.lax.broadcasted_iota(jnp.int32, bcap, 1)
  lane6 = jax.lax.broadcasted_iota(jnp.int32, (bcap[0], 128), 1)

  def step(t, carry):
    sc, first = carry
    m = jnp.max(sc, axis=1, keepdims=True)
    hit = sc == m
    lidx = jnp.where(hit, iota, bcap[1])
    idxm = jnp.min(lidx, axis=1, keepdims=True)
    onehot = jnp.where(iota == idxm, 1.0, 0.0)

    def ext(a):
      return jnp.sum(a * onehot, axis=1, keepdims=True)

    e0 = ext(nb0)
    e1 = ext(nb1)
    e2 = ext(nb2)
    e3 = ext(nb3)
    ec = ext(clsf)

    yy1 = jnp.maximum(e0, nb0)
    xx1 = jnp.maximum(e1, nb1)
    yy2 = jnp.minimum(e2, nb2)
    xx2 = jnp.minimum(e3, nb3)
    inter = jnp.maximum(yy2 - yy1, 0.0) * jnp.maximum(xx2 - xx1, 0.0)
    area_b = (e2 - e0) * (e3 - e1)
    iou = inter / (area_b + areas - inter + 1e-8)
    sc = jnp.where(iou > _IOU_THRESH, -1.0, sc)

    eoff = ec * 4096.0
    cur = jnp.concatenate(
        [e0 - eoff, e1 - eoff, e2 - eoff, e3 - eoff, ec], axis=1)
    first = jnp.where(t == 0, cur, first)
    deg = m < 0.0
    row5 = jnp.where(deg, first, cur)
    score = jnp.where(m > _CONF_THRESH, m, 0.0)
    row = (
        jnp.where(lane6 == 0, row5[:, 0:1], 0.0)
        + jnp.where(lane6 == 1, row5[:, 1:2], 0.0)
        + jnp.where(lane6 == 2, row5[:, 2:3], 0.0)
        + jnp.where(lane6 == 3, row5[:, 3:4], 0.0)
        + jnp.where(lane6 == 4, score, 0.0)
        + jnp.where(lane6 == 5, row5[:, 4:5], 0.0)
    )
    o_ref[pl.ds(t, 1)] = row[None]
    return sc, first

  jax.lax.fori_loop(0, _MAX_DET, step, (sc0, jnp.zeros((bcap[0], 5))),
                    unroll=False)


def _run_nms(fields):
  b = fields.shape[1]
  out = pl.pallas_call(
      _nms_body,
      out_shape=jax.ShapeDtypeStruct((_MAX_DET, b, 128), jnp.float32),
      in_specs=[pl.BlockSpec(memory_space=pltpu.MemorySpace.VMEM)],
      out_specs=pl.BlockSpec(memory_space=pltpu.MemorySpace.VMEM),
  )(fields)
  return out[:, :, :6].transpose(1, 0, 2)


def kernel(class_out, box_out, anchors):
  b, n, c = class_out.shape
  flat = class_out.reshape(b * n * c)
  boxflat = box_out.reshape(b * n * 4)
  ancflat = anchors.reshape(n * 4)
  fields = _sc_select(flat, boxflat, ancflat).reshape(10, _B, _CAP)
  return _run_nms(fields)


def _kernel_xla_topk(class_out, box_out, anchors):
  b, n, c = class_out.shape
  flat = class_out.reshape(b, n * c)
  top_vals, top_idx = jax.lax.top_k(flat, _MAX_DET_POINTS)
  anchor_idx = top_idx // c
  cls_idx = top_idx % c
  box_sel = jnp.take_along_axis(box_out, anchor_idx[..., None], axis=1)
  anchors_sel = jnp.take(anchors, anchor_idx, axis=0)

  cap = 5120
  pad = cap - _MAX_DET_POINTS
  fields = jnp.stack([
      box_sel[..., 0], box_sel[..., 1], box_sel[..., 2], box_sel[..., 3],
      anchors_sel[..., 0], anchors_sel[..., 1], anchors_sel[..., 2],
      anchors_sel[..., 3], top_vals, cls_idx.astype(jnp.float32)
  ])
  padv = jnp.zeros((10, b, pad), jnp.float32).at[8].set(_NEG)
  fields = jnp.concatenate([fields, padv], axis=2)
  return _run_nms(fields)


# speedup vs baseline: 12.7713x; 1.0430x over previous
import functools

import jax
import jax.numpy as jnp
import numpy as np
from jax import lax
from jax.experimental import pallas as pl
from jax.experimental.pallas import tpu as pltpu
from jax.experimental.pallas import tpu_sc as plsc

_NUM_CLASSES = 90
_MAX_DET_POINTS = 5000
_MAX_DET = 100
_IOU_THRESH = 0.5
_CONF_THRESH = 0.3
_BBOX_XFORM_CLIP = float(np.log(1000.0 / 16.0))
_NEG = -1e30

_B = 8
_NPB = 1_800_000
_QN = _NPB // 4
_CHUNK = 18_000
_NCHUNK = _QN // _CHUNK
_VPC = _CHUNK // 16
_HBINS = 8192
_QBINS = _HBINS // 4
_WCAP = 3072
_CAP = 8192
_PCAP = _CAP // 4
_STAGE = 4 * _WCAP
_K = _MAX_DET_POINTS
_MINI32 = np.int32(-2147483648)


def _sc_body(flat, boxflat, anchors, out, inbufa, inbufb, hist, vals, idxs,
             smallv, smalli, cval, cidx, aidxb, aidxa, fbufs,
             hist_sh, red_sh, qtot_sh, tf_sh, cnt_sh, vstage_sh, istage_sh,
             gsem, sema, semb):
  c = lax.axis_index("c")
  s = lax.axis_index("s")
  g = s // 4
  p = s % 4
  batch = c * 4 + g
  wtile = s
  base = batch * _NPB + p * _QN
  iota16 = lax.broadcasted_iota(jnp.int32, (16,), 0)
  ones16 = jnp.ones((16,), jnp.int32)

  def zero_hist(i, _):
    hist[pl.ds(i * 16, 16)] = jnp.zeros((16,), jnp.int32)
    return 0

  lax.fori_loop(0, _HBINS // 16, zero_hist, 0)

  bufs = (inbufa, inbufb)
  sems = (sema, semb)

  def hist_chunk(buf):
    def vloop(i, _):
      v = buf[pl.ds(i * 16, 16)]
      bts = plsc.bitcast(v, jnp.int32)
      key = bts ^ (lax.shift_right_arithmetic(bts, 31) | _MINI32)
      bucket = lax.shift_right_logical(key, 19)
      plsc.addupdate_scatter(hist, [bucket], ones16)
      return 0

    lax.fori_loop(0, _VPC, vloop, 0)

  handles = [None] * _NCHUNK
  handles[0] = pltpu.async_copy(flat.at[pl.ds(base, _CHUNK)], bufs[0], sems[0])
  for ci in range(_NCHUNK):
    if ci + 1 < _NCHUNK:
      handles[ci + 1] = pltpu.async_copy(
          flat.at[pl.ds(base + (ci + 1) * _CHUNK, _CHUNK)],
          bufs[(ci + 1) % 2], sems[(ci + 1) % 2])
    handles[ci].wait()
    hist_chunk(bufs[ci % 2])

  pltpu.sync_copy(hist, hist_sh.at[pl.ds(wtile * _HBINS, _HBINS)])
  plsc.subcore_barrier()

  qbase = p * _QBINS
  pltpu.sync_copy(hist_sh.at[pl.ds(4 * g * _HBINS + qbase, _QBINS)],
                  hist.at[pl.ds(0, _QBINS)])
  for q in range(1, 4):
    pltpu.sync_copy(hist_sh.at[pl.ds((4 * g + q) * _HBINS + qbase, _QBINS)],
                    hist.at[pl.ds(_QBINS, _QBINS)])

    def addq(i, _):
      hist[pl.ds(i * 16, 16)] = (
          hist[pl.ds(i * 16, 16)] + hist[pl.ds(_QBINS + i * 16, 16)])
      return 0

    lax.fori_loop(0, _QBINS // 16, addq, 0)
  pltpu.sync_copy(hist.at[pl.ds(0, _QBINS)],
                  red_sh.at[pl.ds(g * _HBINS + qbase, _QBINS)])

  def qsum(i, a):
    return a + hist[pl.ds(i * 16, 16)]

  part16 = lax.fori_loop(0, _QBINS // 16, qsum, jnp.zeros((16,), jnp.int32))
  qtot = jnp.sum(part16)
  smalli[pl.ds(0, 16)] = jnp.full((16,), qtot, jnp.int32)
  pltpu.sync_copy(smalli.at[pl.ds(0, 128)],
                  qtot_sh.at[pl.ds((g * 4 + p) * 128, 128)])
  plsc.subcore_barrier()

  @pl.when(p == 0)
  def _threshold():
    pltpu.sync_copy(qtot_sh.at[pl.ds(g * 512, 512)], smalli)
    q0 = jnp.max(smalli[pl.ds(0 * 128, 16)])
    q1 = jnp.max(smalli[pl.ds(1 * 128, 16)])
    q2 = jnp.max(smalli[pl.ds(2 * 128, 16)])
    q3 = jnp.max(smalli[pl.ds(3 * 128, 16)])
    c3, c2, c1 = q3, q3 + q2, q3 + q2 + q1
    qq = jnp.where(c3 >= _K, 3, jnp.where(c2 >= _K, 2,
                                          jnp.where(c1 >= _K, 1, 0)))
    above = jnp.where(c3 >= _K, 0, jnp.where(c2 >= _K, q3,
                                             jnp.where(c1 >= _K, c2, c1)))
    pltpu.sync_copy(red_sh.at[pl.ds(g * _HBINS + qq * _QBINS, _QBINS)],
                    hist.at[pl.ds(0, _QBINS)])

    def scan(j, carry):
      accum, bsel, done = carry
      jj = _QBINS // 16 - 1 - j
      v = hist[pl.ds(jj * 16, 16)]
      rv = lax.rev(v, (0,))
      cs = plsc.cumsum(rv)
      tot = jnp.max(cs)
      mask = (accum + cs) >= _K
      found = jnp.logical_and(jnp.logical_not(done), (accum + tot) >= _K)
      lane = jnp.max(plsc.all_reduce_ffs(mask))
      bcand = qq * _QBINS + jj * 16 + 15 - lane
      bsel = jnp.where(found, bcand, bsel)
      done = jnp.logical_or(done, found)
      return accum + tot, bsel, done

    _, bsel, _ = lax.fori_loop(0, _QBINS // 16, scan,
                               (above, jnp.int32(0), False))
    bv = jnp.full((16,), bsel, jnp.int32)
    u = lax.shift_left(bv, 19)
    fb = jnp.where(u < 0, u ^ _MINI32, ~u)
    smallv[pl.ds(0, 16)] = plsc.bitcast(fb, jnp.float32)
    pltpu.sync_copy(smallv, tf_sh.at[pl.ds(g * 128, 128)])

  plsc.subcore_barrier()
  pltpu.sync_copy(tf_sh.at[pl.ds(g * 128, 128)], smallv)
  tv = smallv[pl.ds(0, 16)]

  def compact_chunk(buf, ci, off):
    cbase = p * _QN + ci * _CHUNK

    def vloop(i, off):
      v = buf[pl.ds(i * 16, 16)]
      m = v >= tv
      iv = cbase + i * 16 + iota16
      plsc.store_compressed(vals.at[pl.ds(off, 16)], v, mask=m)
      plsc.store_compressed(idxs.at[pl.ds(off, 16)], iv, mask=m)
      return jnp.minimum(off + jnp.sum(m.astype(jnp.int32)), _WCAP - 160)

    return lax.fori_loop(0, _VPC, vloop, off)

  off = jnp.int32(0)
  handles = [None] * _NCHUNK
  handles[0] = pltpu.async_copy(flat.at[pl.ds(base, _CHUNK)], bufs[0], sems[0])
  for ci in range(_NCHUNK):
    if ci + 1 < _NCHUNK:
      handles[ci + 1] = pltpu.async_copy(
          flat.at[pl.ds(base + (ci + 1) * _CHUNK, _CHUNK)],
          bufs[(ci + 1) % 2], sems[(ci + 1) % 2])
    handles[ci].wait()
    off = compact_chunk(bufs[ci % 2], ci, off)
  for k in range(8):
    vals[pl.ds(off + k * 16, 16)] = jnp.full((16,), _NEG, jnp.float32)
    idxs[pl.ds(off + k * 16, 16)] = jnp.zeros((16,), jnp.int32)
  offr = (off + 127) // 128 * 128
  smalli[pl.ds(0, 16)] = jnp.full((16,), offr, jnp.int32)
  pltpu.sync_copy(smalli.at[pl.ds(0, 128)],
                  cnt_sh.at[pl.ds((g * 4 + p) * 128, 128)])
  plsc.subcore_barrier()

  pltpu.sync_copy(cnt_sh.at[pl.ds(g * 512, 512)], smalli)
  n0 = jnp.max(smalli[pl.ds(0 * 128, 16)])
  n1 = jnp.max(smalli[pl.ds(1 * 128, 16)])
  n2 = jnp.max(smalli[pl.ds(2 * 128, 16)])
  n3 = jnp.max(smalli[pl.ds(3 * 128, 16)])
  myoff = jnp.where(p == 0, 0, jnp.where(p == 1, n0,
                                         jnp.where(p == 2, n0 + n1,
                                                   n0 + n1 + n2)))
  mr = n0 + n1 + n2 + n3

  for ph in range(4):
    @pl.when(p == ph)
    def _stage():
      moff = pl.multiple_of(g * _STAGE + myoff, 128)
      pltpu.sync_copy(vals, vstage_sh.at[pl.ds(moff, _WCAP)])
      pltpu.sync_copy(idxs, istage_sh.at[pl.ds(moff, _WCAP)])

    plsc.subcore_barrier()

  cb = p * _PCAP
  pltpu.sync_copy(vstage_sh.at[pl.ds(g * _STAGE + cb, _PCAP)], cval)
  pltpu.sync_copy(istage_sh.at[pl.ds(g * _STAGE + cb, _PCAP)], cidx)
  rowoff = batch * 20000

  def prep(k, _):
    pos = cb + k * 16 + iota16
    valid = pos < mr
    v = jnp.where(valid, cval[pl.ds(k * 16, 16)], _NEG)
    fi = jnp.where(valid, cidx[pl.ds(k * 16, 16)], 0)
    a = fi // _NUM_CLASSES
    cls = fi - a * _NUM_CLASSES
    for cdim in range(4):
      aidxb[pl.ds(cdim * _PCAP + k * 16, 16)] = (a + rowoff) * 4 + cdim
      aidxa[pl.ds(cdim * _PCAP + k * 16, 16)] = a * 4 + cdim
    fbufs[pl.ds(8 * _PCAP + k * 16, 16)] = v
    fbufs[pl.ds(9 * _PCAP + k * 16, 16)] = cls.astype(jnp.float32)
    return 0

  lax.fori_loop(0, _PCAP // 16, prep, 0)

  cops = []
  for cdim in range(4):
    for j in range(_PCAP // 128):
      o = cdim * _PCAP + j * 128
      cops.append(pltpu.async_copy(
          boxflat.at[aidxb.at[pl.ds(o, 128)]],
          fbufs.at[pl.ds(cdim * _PCAP + j * 128, 128)], gsem))
      cops.append(pltpu.async_copy(
          anchors.at[aidxa.at[pl.ds(o, 128)]],
          fbufs.at[pl.ds((4 + cdim) * _PCAP + j * 128, 128)], gsem))
  for cp in cops:
    cp.wait()

  for f in range(10):
    pltpu.sync_copy(
        fbufs.at[pl.ds(f * _PCAP, _PCAP)],
        out.at[pl.ds(f * (_B * _CAP) + batch * _CAP + cb, _PCAP)])


def _sc_select(flat, boxflat, anchors):
  mesh = plsc.VectorSubcoreMesh(core_axis_name="c", subcore_axis_name="s",
                                num_cores=2, num_subcores=16)
  return pl.kernel(
      _sc_body,
      out_type=jax.ShapeDtypeStruct((10 * _B * _CAP,), jnp.float32),
      mesh=mesh,
      compiler_params=pltpu.CompilerParams(needs_layout_passes=False),
      scratch_types=[
          pltpu.VMEM((_CHUNK,), jnp.float32),
          pltpu.VMEM((_CHUNK,), jnp.float32),
          pltpu.VMEM((_HBINS,), jnp.int32),
          pltpu.VMEM((_WCAP,), jnp.float32),
          pltpu.VMEM((_WCAP,), jnp.int32),
          pltpu.VMEM((128,), jnp.float32),
          pltpu.VMEM((512,), jnp.int32),
          pltpu.VMEM((_PCAP,), jnp.float32),
          pltpu.VMEM((_PCAP,), jnp.int32),
          pltpu.VMEM((4 * _PCAP,), jnp.int32),
          pltpu.VMEM((4 * _PCAP,), jnp.int32),
          pltpu.VMEM((10 * _PCAP,), jnp.float32),
          pltpu.VMEM_SHARED((16 * _HBINS,), jnp.int32),
          pltpu.VMEM_SHARED((4 * _HBINS,), jnp.int32),
          pltpu.VMEM_SHARED((2048,), jnp.int32),
          pltpu.VMEM_SHARED((512,), jnp.float32),
          pltpu.VMEM_SHARED((2048,), jnp.int32),
          pltpu.VMEM_SHARED((4 * _STAGE,), jnp.float32),
          pltpu.VMEM_SHARED((4 * _STAGE,), jnp.int32),
          pltpu.SemaphoreType.DMA,
          pltpu.SemaphoreType.DMA,
          pltpu.SemaphoreType.DMA,
      ],
  )(flat, boxflat, anchors)


def _nms_body(f_ref, o_ref):
  ty = f_ref[0]
  tx = f_ref[1]
  th = jnp.minimum(f_ref[2], _BBOX_XFORM_CLIP)
  tw = jnp.minimum(f_ref[3], _BBOX_XFORM_CLIP)
  ay1 = f_ref[4]
  ax1 = f_ref[5]
  ay2 = f_ref[6]
  ax2 = f_ref[7]
  logit = f_ref[8]
  clsf = f_ref[9]

  yca = (ay1 + ay2) / 2.0
  xca = (ax1 + ax2) / 2.0
  ha = ay2 - ay1
  wa = ax2 - ax1
  yc = ty * ha + yca
  xc = tx * wa + xca
  h = jnp.exp(th) * ha
  w = jnp.exp(tw) * wa
  off = clsf * 4096.0
  nb0 = (yc - h / 2.0) + off
  nb1 = (xc - w / 2.0) + off
  nb2 = (yc + h / 2.0) + off
  nb3 = (xc + w / 2.0) + off
  areas = (nb2 - nb0) * (nb3 - nb1)
  sc0 = jnp.where(logit > 0.5 * _NEG, jax.nn.sigmoid(logit), -1.0)

  bcap = sc0.shape
  iota = jax.lax.broadcasted_iota(jnp.int32, bcap, 1)
  lane6 = jax.lax.broadcasted_iota(jnp.int32, (bcap[0], 128), 1)

  def step(t, carry):
    sc, first = carry
    m = jnp.max(sc, axis=1, keepdims=True)
    hit = sc == m
    lidx = jnp.where(hit, iota, bcap[1])
    idxm = jnp.min(lidx, axis=1, keepdims=True)
    onehot = jnp.where(iota == idxm, 1.0, 0.0)

    def ext(a):
      return jnp.sum(a * onehot, axis=1, keepdims=True)

    e0 = ext(nb0)
    e1 = ext(nb1)
    e2 = ext(nb2)
    e3 = ext(nb3)
    ec = ext(clsf)

    yy1 = jnp.maximum(e0, nb0)
    xx1 = jnp.maximum(e1, nb1)
    yy2 = jnp.minimum(e2, nb2)
    xx2 = jnp.minimum(e3, nb3)
    inter = jnp.maximum(yy2 - yy1, 0.0) * jnp.maximum(xx2 - xx1, 0.0)
    area_b = (e2 - e0) * (e3 - e1)
    iou = inter / (area_b + areas - inter + 1e-8)
    sc = jnp.where(iou > _IOU_THRESH, -1.0, sc)

    eoff = ec * 4096.0
    cur = jnp.concatenate(
        [e0 - eoff, e1 - eoff, e2 - eoff, e3 - eoff, ec], axis=1)
    first = jnp.where(t == 0, cur, first)
    deg = m < 0.0
    row5 = jnp.where(deg, first, cur)
    score = jnp.where(m > _CONF_THRESH, m, 0.0)
    row = (
        jnp.where(lane6 == 0, row5[:, 0:1], 0.0)
        + jnp.where(lane6 == 1, row5[:, 1:2], 0.0)
        + jnp.where(lane6 == 2, row5[:, 2:3], 0.0)
        + jnp.where(lane6 == 3, row5[:, 3:4], 0.0)
        + jnp.where(lane6 == 4, score, 0.0)
        + jnp.where(lane6 == 5, row5[:, 4:5], 0.0)
    )
    o_ref[pl.ds(t, 1)] = row[None]
    return sc, first

  jax.lax.fori_loop(0, _MAX_DET, step, (sc0, jnp.zeros((bcap[0], 5))),
                    unroll=False)


def _run_nms(fields):
  b = fields.shape[1]
  out = pl.pallas_call(
      _nms_body,
      out_shape=jax.ShapeDtypeStruct((_MAX_DET, b, 128), jnp.float32),
      in_specs=[pl.BlockSpec(memory_space=pltpu.MemorySpace.VMEM)],
      out_specs=pl.BlockSpec(memory_space=pltpu.MemorySpace.VMEM),
  )(fields)
  return out[:, :, :6].transpose(1, 0, 2)


def kernel(class_out, box_out, anchors):
  b, n, c = class_out.shape
  flat = class_out.reshape(b * n * c)
  boxflat = box_out.reshape(b * n * 4)
  ancflat = anchors.reshape(n * 4)
  fields = _sc_select(flat, boxflat, ancflat).reshape(10, _B, _CAP)
  return _run_nms(fields)


def _kernel_xla_topk(class_out, box_out, anchors):
  b, n, c = class_out.shape
  flat = class_out.reshape(b, n * c)
  top_vals, top_idx = jax.lax.top_k(flat, _MAX_DET_POINTS)
  anchor_idx = top_idx // c
  cls_idx = top_idx % c
  box_sel = jnp.take_along_axis(box_out, anchor_idx[..., None], axis=1)
  anchors_sel = jnp.take(anchors, anchor_idx, axis=0)

  cap = 5120
  pad = cap - _MAX_DET_POINTS
  fields = jnp.stack([
      box_sel[..., 0], box_sel[..., 1], box_sel[..., 2], box_sel[..., 3],
      anchors_sel[..., 0], anchors_sel[..., 1], anchors_sel[..., 2],
      anchors_sel[..., 3], top_vals, cls_idx.astype(jnp.float32)
  ])
  padv = jnp.zeros((10, b, pad), jnp.float32).at[8].set(_NEG)
  fields = jnp.concatenate([fields, padv], axis=2)
  return _run_nms(fields)
